# baseline (device time: 49909 ns/iter reference)
import functools

import jax
import jax.numpy as jnp
from jax import lax
from jax.experimental import pallas as pl
from jax.experimental.pallas import tpu as pltpu

N_DEV = 16
ROUNDS = 4
B, SQ, SKV, HQ_LOC, DH = 2, 128, 128, 4, 64
D_MODEL = 512
D_LOC = HQ_LOC * DH


def kernel(x, Wq, K_ext, V_ext, Wo):
    my_i = lax.axis_index("i")
    Kl = lax.dynamic_slice_in_dim(K_ext, my_i * HQ_LOC, HQ_LOC, axis=2)
    Vl = lax.dynamic_slice_in_dim(V_ext, my_i * HQ_LOC, HQ_LOC, axis=2)
    Kl = jnp.transpose(Kl, (0, 2, 1, 3)).reshape(B * HQ_LOC, SKV, DH)
    Vl = jnp.transpose(Vl, (0, 2, 1, 3)).reshape(B * HQ_LOC, SKV, DH)
    x2d = x.reshape(B * SQ, D_MODEL)

    def body(x_ref, wq_ref, k_ref, v_ref, wo_ref, out_ref,
             ctx_ref, acc_ref, recv_ref, send_sems, recv_sems):
        my = lax.axis_index("i")

        barrier = pltpu.get_barrier_semaphore()
        for r in range(ROUNDS):
            partner = my ^ (1 << r)
            pl.semaphore_signal(
                barrier, inc=1,
                device_id=(partner,), device_id_type=pl.DeviceIdType.MESH,
            )
        pl.semaphore_wait(barrier, ROUNDS)

        q2d = lax.dot_general(
            x_ref[...], wq_ref[...], (((1,), (0,)), ((), ())),
            preferred_element_type=jnp.float32,
        )
        for b in range(B):
            for h in range(HQ_LOC):
                q_bh = q2d[b * SQ:(b + 1) * SQ, h * DH:(h + 1) * DH]
                k_bh = k_ref[b * HQ_LOC + h]
                v_bh = v_ref[b * HQ_LOC + h]
                s = lax.dot_general(
                    q_bh, k_bh, (((1,), (1,)), ((), ())),
                    preferred_element_type=jnp.float32,
                ) * 0.125
                m = jnp.max(s, axis=1, keepdims=True)
                w = jnp.exp(s - m)
                w = w / jnp.sum(w, axis=1, keepdims=True)
                ctx_ref[b * SQ:(b + 1) * SQ, h * DH:(h + 1) * DH] = (
                    lax.dot_general(
                        w, v_bh, (((1,), (0,)), ((), ())),
                        preferred_element_type=jnp.float32,
                    )
                )
        acc_ref[...] = lax.dot_general(
            ctx_ref[...], wo_ref[...], (((1,), (0,)), ((), ())),
            preferred_element_type=jnp.float32,
        )

        for r in range(ROUNDS):
            partner = my ^ (1 << r)
            rdma = pltpu.make_async_remote_copy(
                src_ref=acc_ref,
                dst_ref=recv_ref.at[r],
                send_sem=send_sems.at[r],
                recv_sem=recv_sems.at[r],
                device_id=(partner,),
                device_id_type=pl.DeviceIdType.MESH,
            )
            rdma.start()
            rdma.wait()
            acc_ref[...] = acc_ref[...] + recv_ref[r]
        out_ref[...] = acc_ref[...]

        @functools.partial(pl.run_scoped, exit_sem=pltpu.SemaphoreType.REGULAR)
        def _(exit_sem):
            for r in range(ROUNDS):
                partner = my ^ (1 << r)
                pl.semaphore_signal(
                    exit_sem, inc=1,
                    device_id=(partner,), device_id_type=pl.DeviceIdType.MESH,
                )
            pl.semaphore_wait(exit_sem, ROUNDS)

    out2d = pl.pallas_call(
        body,
        out_shape=jax.ShapeDtypeStruct((B * SQ, D_MODEL), jnp.float32),
        in_specs=[pl.BlockSpec(memory_space=pltpu.VMEM)] * 5,
        out_specs=pl.BlockSpec(memory_space=pltpu.VMEM),
        scratch_shapes=[
            pltpu.VMEM((B * SQ, D_LOC), jnp.float32),
            pltpu.VMEM((B * SQ, D_MODEL), jnp.float32),
            pltpu.VMEM((ROUNDS, B * SQ, D_MODEL), jnp.float32),
            pltpu.SemaphoreType.DMA((ROUNDS,)),
            pltpu.SemaphoreType.DMA((ROUNDS,)),
        ],
        compiler_params=pltpu.CompilerParams(collective_id=0),
    )(x2d, Wq, Kl, Vl, Wo)
    return out2d.reshape(B, SQ, D_MODEL)


# device time: 35804 ns/iter; 1.3940x vs baseline; 1.3940x over previous
import functools

import jax
import jax.numpy as jnp
from jax import lax
from jax.experimental import pallas as pl
from jax.experimental.pallas import tpu as pltpu

N_DEV = 16
ROUNDS = 4
B, SQ, SKV, HQ_LOC, DH = 2, 128, 128, 4, 64
D_MODEL = 512
D_LOC = HQ_LOC * DH
D_HALF = D_MODEL // 2

MASKS_A = (1, 3, 4, 8)
MASKS_B = (4, 8, 1, 3)


def kernel(x, Wq, K_ext, V_ext, Wo):
    my_i = lax.axis_index("i")
    Kl = lax.dynamic_slice_in_dim(K_ext, my_i * HQ_LOC, HQ_LOC, axis=2)
    Vl = lax.dynamic_slice_in_dim(V_ext, my_i * HQ_LOC, HQ_LOC, axis=2)
    Kl = jnp.transpose(Kl, (0, 2, 1, 3)).reshape(B * HQ_LOC, SKV, DH)
    Vl = jnp.transpose(Vl, (0, 2, 1, 3)).reshape(B * HQ_LOC, SKV, DH)
    x2d = x.reshape(B * SQ, D_MODEL)

    def body(x_ref, wq_ref, k_ref, v_ref, wo_ref, out_ref,
             ctx_ref, acc_ref, recv_ref, send_sems, recv_sems):
        my = lax.axis_index("i")

        barrier = pltpu.get_barrier_semaphore()
        for m in MASKS_A:
            partner = my ^ m
            pl.semaphore_signal(
                barrier, inc=1,
                device_id=(partner,), device_id_type=pl.DeviceIdType.MESH,
            )
        pl.semaphore_wait(barrier, ROUNDS)

        q2d = lax.dot_general(
            x_ref[...], wq_ref[...], (((1,), (0,)), ((), ())),
            preferred_element_type=jnp.float32,
        )
        for b in range(B):
            for h in range(HQ_LOC):
                q_bh = q2d[b * SQ:(b + 1) * SQ, h * DH:(h + 1) * DH]
                k_bh = k_ref[b * HQ_LOC + h]
                v_bh = v_ref[b * HQ_LOC + h]
                s = lax.dot_general(
                    q_bh, k_bh, (((1,), (1,)), ((), ())),
                    preferred_element_type=jnp.float32,
                ) * 0.125
                m = jnp.max(s, axis=1, keepdims=True)
                w = jnp.exp(s - m)
                w = w / jnp.sum(w, axis=1, keepdims=True)
                ctx_ref[b * SQ:(b + 1) * SQ, h * DH:(h + 1) * DH] = (
                    lax.dot_general(
                        w, v_bh, (((1,), (0,)), ((), ())),
                        preferred_element_type=jnp.float32,
                    )
                )
        ctx = ctx_ref[...]
        wo = wo_ref[...]
        for s in range(2):
            acc_ref[s] = lax.dot_general(
                ctx, wo[:, s * D_HALF:(s + 1) * D_HALF],
                (((1,), (0,)), ((), ())),
                preferred_element_type=jnp.float32,
            )

        masks = (MASKS_A, MASKS_B)

        def make_rdma(s, r):
            return pltpu.make_async_remote_copy(
                src_ref=acc_ref.at[s],
                dst_ref=recv_ref.at[s, r],
                send_sem=send_sems.at[s, r],
                recv_sem=recv_sems.at[s, r],
                device_id=(my ^ masks[s][r],),
                device_id_type=pl.DeviceIdType.MESH,
            )

        make_rdma(0, 0).start()
        make_rdma(1, 0).start()
        for r in range(ROUNDS):
            for s in range(2):
                rdma = make_rdma(s, r)
                rdma.wait()
                acc_ref[s] = acc_ref[s] + recv_ref[s, r]
                if r + 1 < ROUNDS:
                    make_rdma(s, r + 1).start()
        for s in range(2):
            out_ref[:, s * D_HALF:(s + 1) * D_HALF] = acc_ref[s]

        @functools.partial(pl.run_scoped, exit_sem=pltpu.SemaphoreType.REGULAR)
        def _(exit_sem):
            for m in MASKS_A:
                partner = my ^ m
                pl.semaphore_signal(
                    exit_sem, inc=1,
                    device_id=(partner,), device_id_type=pl.DeviceIdType.MESH,
                )
            pl.semaphore_wait(exit_sem, ROUNDS)

    out2d = pl.pallas_call(
        body,
        out_shape=jax.ShapeDtypeStruct((B * SQ, D_MODEL), jnp.float32),
        in_specs=[pl.BlockSpec(memory_space=pltpu.VMEM)] * 5,
        out_specs=pl.BlockSpec(memory_space=pltpu.VMEM),
        scratch_shapes=[
            pltpu.VMEM((B * SQ, D_LOC), jnp.float32),
            pltpu.VMEM((2, B * SQ, D_HALF), jnp.float32),
            pltpu.VMEM((2, ROUNDS, B * SQ, D_HALF), jnp.float32),
            pltpu.SemaphoreType.DMA((2, ROUNDS)),
            pltpu.SemaphoreType.DMA((2, ROUNDS)),
        ],
        compiler_params=pltpu.CompilerParams(collective_id=0),
    )(x2d, Wq, Kl, Vl, Wo)
    return out2d.reshape(B, SQ, D_MODEL)
